# baseline (speedup 1.0000x reference)
import functools
import math

import jax
import jax.numpy as jnp
from jax import lax
from jax.experimental import pallas as pl
from jax.experimental.pallas import tpu as pltpu
from jax.experimental.pallas import tpu_sc as plsc

B, G, E = 64, 2048, 384
K = G // 2
V = G - K
NW = 32
ROWS_PER_W = B // NW
GCHUNK = 64
NCHUNK = V // GCHUNK
NBUF = 4


def _posmap_body(scores_ref, posmap_ref):
    s = scores_ref[...]
    nr = s.shape[0]
    bits = lax.bitcast_convert_type(s, jnp.int32)

    def step(_, carry):
        lo, hi = carry
        mid = lo + ((hi - lo) >> 1)
        cnt = jnp.sum((bits > mid).astype(jnp.int32), axis=1, keepdims=True)
        pred = cnt < K
        return jnp.where(pred, lo, mid + 1), jnp.where(pred, mid, hi)

    lo0 = jnp.zeros((nr, 1), jnp.int32)
    hi0 = jnp.full((nr, 1), 0x3F800000, jnp.int32)
    t, _ = lax.fori_loop(0, 30, step, (lo0, hi0))

    gt = (bits > t).astype(jnp.int32)
    cnt_gt = jnp.sum(gt, axis=1, keepdims=True)
    need = K - cnt_gt
    tie = (bits == t).astype(jnp.int32)
    x2 = jnp.concatenate([gt.astype(jnp.bfloat16),
                          tie.astype(jnp.bfloat16)], axis=0)
    i0 = lax.broadcasted_iota(jnp.int32, (G, G), 0)
    i1 = lax.broadcasted_iota(jnp.int32, (G, G), 1)
    m = (i0 <= i1).astype(jnp.bfloat16)
    cs = lax.dot_general(x2, m, (((1,), (0,)), ((), ())),
                         preferred_element_type=jnp.float32)
    c_gt = cs[:nr].astype(jnp.int32)
    c_tie = cs[nr:].astype(jnp.int32)
    masked = (gt == 1) | ((tie == 1) & (c_tie - tie < need))
    cmask = c_gt + jnp.minimum(c_tie, need)
    gidx = lax.broadcasted_iota(jnp.int32, (nr, G), 1)
    posmap_ref[...] = jnp.where(masked, -1, gidx - cmask)


HALF = B // 2


def _posmap_half(rand_scores, h):
    return pl.pallas_call(
        _posmap_body,
        grid=(1,),
        in_specs=[pl.BlockSpec((HALF, G), lambda i: (h, 0))],
        out_specs=pl.BlockSpec((HALF, G), lambda i: (0, 0)),
        out_shape=jax.ShapeDtypeStruct((HALF, G), jnp.int32),
    )(rand_scores)


def _sc_mesh():
    return plsc.VectorSubcoreMesh(core_axis_name="c", subcore_axis_name="s",
                                  num_cores=2, num_subcores=16)


def _make_compact_body(offset):
    def body(posmap_hbm, coords_hbm, out_idx_hbm, out_coords_hbm,
             pos_v, coords_v, idx_v, cvis_v):
        wid = lax.axis_index("s") * 2 + lax.axis_index("c")
        b = wid
        bg = offset + wid
        pltpu.sync_copy(posmap_hbm.at[b], pos_v)
        pltpu.sync_copy(coords_hbm.at[bg], coords_v)

        base_flat = bg * G
        UNROLL = 4

        def chunk(ci, _):
            for u in range(UNROLL):
                off = ci * (16 * UNROLL) + u * 16
                pos = pos_v[pl.ds(off, 16)]
                m = pos >= 0
                g = off + lax.iota(jnp.int32, 16)
                plsc.store_scatter(idx_v, [pos], base_flat + g, mask=m)
                cvals = coords_v[pl.ds(off, 16)]
                plsc.store_scatter(cvis_v, [pos], cvals, mask=m)
            return 0

        lax.fori_loop(0, G // (16 * UNROLL), chunk, 0)

        pltpu.sync_copy(idx_v, out_idx_hbm.at[b])
        pltpu.sync_copy(cvis_v, out_coords_hbm.at[pl.ds(b * V, V)])
    return body


def _sc_gather_body(idx1_hbm, idx2_hbm, feats_hbm, out_feats_hbm,
                    idx_v, rows_v, sems_g, sems_w):
    wid = lax.axis_index("s") * 2 + lax.axis_index("c")

    for r in range(ROWS_PER_W):
        idx_hbm = idx1_hbm if r == 0 else idx2_hbm
        b = wid + r * HALF
        pltpu.sync_copy(idx_hbm.at[wid], idx_v)

        gathers = [None] * NBUF
        writes = [None] * NBUF
        for ch in range(NCHUNK):
            k = ch % NBUF
            if writes[k] is not None:
                writes[k].wait()
                writes[k] = None
            sl = idx_v.at[pl.ds(ch * GCHUNK, GCHUNK)]
            gathers[k] = pltpu.async_copy(
                feats_hbm.at[sl], rows_v.at[k], sems_g.at[k])
            if ch > 0:
                p = (ch - 1) % NBUF
                gathers[p].wait()
                writes[p] = pltpu.async_copy(
                    rows_v.at[p],
                    out_feats_hbm.at[b, pl.ds((ch - 1) * GCHUNK, GCHUNK)],
                    sems_w.at[p])
        last = (NCHUNK - 1) % NBUF
        gathers[last].wait()
        writes[last] = pltpu.async_copy(
            rows_v.at[last],
            out_feats_hbm.at[b, pl.ds((NCHUNK - 1) * GCHUNK, GCHUNK)],
            sems_w.at[last])
        for k in range(NBUF):
            if writes[k] is not None:
                writes[k].wait()


def _sc_compact(posmap_half, coords2d, offset):
    kern = pl.kernel(
        _make_compact_body(offset),
        out_type=[
            jax.ShapeDtypeStruct((HALF, V), jnp.int32),
            jax.ShapeDtypeStruct((HALF * V,), jnp.float32),
        ],
        mesh=_sc_mesh(),
        compiler_params=pltpu.CompilerParams(needs_layout_passes=False),
        scratch_types=[
            pltpu.VMEM((G,), jnp.int32),
            pltpu.VMEM((G,), jnp.float32),
            pltpu.VMEM((V,), jnp.int32),
            pltpu.VMEM((V,), jnp.float32),
        ],
    )
    return kern(posmap_half, coords2d)


def _sc_gather(idx1, idx2, feats_flat):
    kern = pl.kernel(
        _sc_gather_body,
        out_type=jax.ShapeDtypeStruct((B, V, E), jnp.float32),
        mesh=_sc_mesh(),
        compiler_params=pltpu.CompilerParams(needs_layout_passes=False),
        scratch_types=[
            pltpu.VMEM((V,), jnp.int32),
            pltpu.VMEM((NBUF, GCHUNK, E), jnp.float32),
            pltpu.SemaphoreType.DMA((NBUF,)),
            pltpu.SemaphoreType.DMA((NBUF,)),
        ],
    )
    return kern(idx1, idx2, feats_flat)


RBLK = 8


def _mlp_body(cv_ref, w1_ref, b1_ref, w2_ref, b2_ref, g_ref, beta_ref,
              out_ref):
    w1c = jnp.transpose(w1_ref[...])
    b1c = jnp.transpose(b1_ref[...].reshape(1, 128))
    w2 = w2_ref[...]
    b2 = b2_ref[...].reshape(1, E)
    gam = g_ref[...].reshape(1, E)
    bet = beta_ref[...].reshape(1, E)
    for s in range(RBLK):
        c_row = cv_ref[s:s + 1, :]
        ht = w1c * c_row + b1c
        ht = 0.5 * ht * (1.0 + lax.erf(ht * (1.0 / math.sqrt(2.0))))
        h2 = lax.dot_general(ht, w2, (((0,), (0,)), ((), ())),
                             preferred_element_type=jnp.float32)
        h2 = h2 + b2
        mean = jnp.mean(h2, axis=1, keepdims=True)
        ctr = h2 - mean
        var = jnp.mean(ctr * ctr, axis=1, keepdims=True)
        out_ref[pl.ds(s * V, V), :] = (
            ctr / jnp.sqrt(var + 1e-5) * gam + bet)


def _mlp(cv, W1, b1, W2, b2, ln_gamma, ln_beta):
    nt = B * V
    return pl.pallas_call(
        _mlp_body,
        grid=(B // RBLK,),
        in_specs=[
            pl.BlockSpec((RBLK, V), lambda i: (i, 0)),
            pl.BlockSpec((1, 128), lambda i: (0, 0)),
            pl.BlockSpec((128,), lambda i: (0,)),
            pl.BlockSpec((128, E), lambda i: (0, 0)),
            pl.BlockSpec((E,), lambda i: (0,)),
            pl.BlockSpec((E,), lambda i: (0,)),
            pl.BlockSpec((E,), lambda i: (0,)),
        ],
        out_specs=pl.BlockSpec((RBLK * V, E), lambda i: (i, 0)),
        out_shape=jax.ShapeDtypeStruct((nt, E), jnp.float32),
    )(cv, W1, b1, W2, b2, ln_gamma, ln_beta)


def kernel(feats_emb, center_coords, rand_scores, W1, b1, W2, b2,
           ln_gamma, ln_beta):
    coords2d = center_coords.reshape(B, G)
    feats_flat = feats_emb.reshape(B * G, E)
    pm1 = _posmap_half(rand_scores, 0)
    idx1, cv1 = _sc_compact(pm1, coords2d, 0)
    pm2 = _posmap_half(rand_scores, 1)
    idx2, cv2 = _sc_compact(pm2, coords2d, HALF)
    fv = _sc_gather(idx1, idx2, feats_flat)
    cvis = jnp.concatenate([cv1, cv2]).reshape(B, V)
    pos_emb = _mlp(cvis, W1, b1, W2, b2, ln_gamma, ln_beta)
    return fv, pos_emb.reshape(B, V, E)


# speedup vs baseline: 1.0268x; 1.0268x over previous
import functools
import math

import jax
import jax.numpy as jnp
from jax import lax
from jax.experimental import pallas as pl
from jax.experimental.pallas import tpu as pltpu
from jax.experimental.pallas import tpu_sc as plsc

B, G, E = 64, 2048, 384
K = G // 2
V = G - K
NW = 32
ROWS_PER_W = B // NW
GCHUNK = 64
NCHUNK = V // GCHUNK
NBUF = 4


def _posmap_body(scores_ref, posmap_ref):
    s = scores_ref[...]
    nr = s.shape[0]
    bits = lax.bitcast_convert_type(s, jnp.int32)

    def step(_, carry):
        lo, hi = carry
        mid = lo + ((hi - lo) >> 1)
        cnt = jnp.sum((bits > mid).astype(jnp.int32), axis=1, keepdims=True)
        pred = cnt < K
        return jnp.where(pred, lo, mid + 1), jnp.where(pred, mid, hi)

    lo0 = jnp.zeros((nr, 1), jnp.int32)
    hi0 = jnp.full((nr, 1), 0x3F800000, jnp.int32)
    t, _ = lax.fori_loop(0, 30, step, (lo0, hi0))

    gt = (bits > t).astype(jnp.int32)
    cnt_gt = jnp.sum(gt, axis=1, keepdims=True)
    need = K - cnt_gt
    tie = (bits == t).astype(jnp.int32)
    x2 = jnp.concatenate([gt.astype(jnp.bfloat16),
                          tie.astype(jnp.bfloat16)], axis=0)
    i0 = lax.broadcasted_iota(jnp.int32, (G, G), 0)
    i1 = lax.broadcasted_iota(jnp.int32, (G, G), 1)
    m = (i0 <= i1).astype(jnp.bfloat16)
    cs = lax.dot_general(x2, m, (((1,), (0,)), ((), ())),
                         preferred_element_type=jnp.float32)
    c_gt = cs[:nr].astype(jnp.int32)
    c_tie = cs[nr:].astype(jnp.int32)
    masked = (gt == 1) | ((tie == 1) & (c_tie - tie < need))
    cmask = c_gt + jnp.minimum(c_tie, need)
    gidx = lax.broadcasted_iota(jnp.int32, (nr, G), 1)
    posmap_ref[...] = jnp.where(masked, -1, gidx - cmask)


def _posmap(rand_scores):
    return pl.pallas_call(
        _posmap_body,
        out_shape=jax.ShapeDtypeStruct((B, G), jnp.int32),
    )(rand_scores)


def _sc_mesh():
    return plsc.VectorSubcoreMesh(core_axis_name="c", subcore_axis_name="s",
                                  num_cores=2, num_subcores=16)


def _sc_compact_body(posmap_hbm, coords_hbm, out_idx_hbm, out_coords_hbm,
                     pos_v, coords_v, idx_v, cvis_v):
    wid = lax.axis_index("s") * 2 + lax.axis_index("c")

    for r in range(ROWS_PER_W):
        b = wid * ROWS_PER_W + r
        pltpu.sync_copy(posmap_hbm.at[b], pos_v)
        pltpu.sync_copy(coords_hbm.at[b], coords_v)

        base_flat = b * G
        UNROLL = 4

        def chunk(ci, _):
            for u in range(UNROLL):
                off = ci * (16 * UNROLL) + u * 16
                pos = pos_v[pl.ds(off, 16)]
                m = pos >= 0
                g = off + lax.iota(jnp.int32, 16)
                plsc.store_scatter(idx_v, [pos], base_flat + g, mask=m)
                cvals = coords_v[pl.ds(off, 16)]
                plsc.store_scatter(cvis_v, [pos], cvals, mask=m)
            return 0

        lax.fori_loop(0, G // (16 * UNROLL), chunk, 0)

        pltpu.sync_copy(idx_v, out_idx_hbm.at[b])
        pltpu.sync_copy(cvis_v, out_coords_hbm.at[pl.ds(b * V, V)])


def _sc_gather_body(idx_hbm, feats_hbm, out_feats_hbm,
                    idx_v, rows_v, sems_g, sems_w):
    wid = lax.axis_index("s") * 2 + lax.axis_index("c")

    for r in range(ROWS_PER_W):
        b = wid * ROWS_PER_W + r
        pltpu.sync_copy(idx_hbm.at[b], idx_v)

        gathers = [None] * NBUF
        writes = [None] * NBUF
        for ch in range(NCHUNK):
            k = ch % NBUF
            if writes[k] is not None:
                writes[k].wait()
                writes[k] = None
            sl = idx_v.at[pl.ds(ch * GCHUNK, GCHUNK)]
            gathers[k] = pltpu.async_copy(
                feats_hbm.at[sl], rows_v.at[k], sems_g.at[k])
            if ch > 0:
                p = (ch - 1) % NBUF
                gathers[p].wait()
                writes[p] = pltpu.async_copy(
                    rows_v.at[p],
                    out_feats_hbm.at[b, pl.ds((ch - 1) * GCHUNK, GCHUNK)],
                    sems_w.at[p])
        last = (NCHUNK - 1) % NBUF
        gathers[last].wait()
        writes[last] = pltpu.async_copy(
            rows_v.at[last],
            out_feats_hbm.at[b, pl.ds((NCHUNK - 1) * GCHUNK, GCHUNK)],
            sems_w.at[last])
        for k in range(NBUF):
            if writes[k] is not None:
                writes[k].wait()


def _sc_compact(posmap, coords2d):
    kern = pl.kernel(
        _sc_compact_body,
        out_type=[
            jax.ShapeDtypeStruct((B, V), jnp.int32),
            jax.ShapeDtypeStruct((B * V,), jnp.float32),
        ],
        mesh=_sc_mesh(),
        compiler_params=pltpu.CompilerParams(needs_layout_passes=False),
        scratch_types=[
            pltpu.VMEM((G,), jnp.int32),
            pltpu.VMEM((G,), jnp.float32),
            pltpu.VMEM((V,), jnp.int32),
            pltpu.VMEM((V,), jnp.float32),
        ],
    )
    return kern(posmap, coords2d)


def _sc_gather(idx_all, feats_flat):
    kern = pl.kernel(
        _sc_gather_body,
        out_type=jax.ShapeDtypeStruct((B, V, E), jnp.float32),
        mesh=_sc_mesh(),
        compiler_params=pltpu.CompilerParams(needs_layout_passes=False),
        scratch_types=[
            pltpu.VMEM((V,), jnp.int32),
            pltpu.VMEM((NBUF, GCHUNK, E), jnp.float32),
            pltpu.SemaphoreType.DMA((NBUF,)),
            pltpu.SemaphoreType.DMA((NBUF,)),
        ],
    )
    return kern(idx_all, feats_flat)


RBLK = 8


def _mlp_body(cv_ref, w1_ref, b1_ref, w2_ref, b2_ref, g_ref, beta_ref,
              out_ref):
    w1c = jnp.transpose(w1_ref[...])
    b1c = jnp.transpose(b1_ref[...].reshape(1, 128))
    w2 = w2_ref[...]
    b2 = b2_ref[...].reshape(1, E)
    gam = g_ref[...].reshape(1, E)
    bet = beta_ref[...].reshape(1, E)
    for s in range(RBLK):
        c_row = cv_ref[s:s + 1, :]
        ht = w1c * c_row + b1c
        ht = 0.5 * ht * (1.0 + lax.erf(ht * (1.0 / math.sqrt(2.0))))
        h2 = lax.dot_general(ht, w2, (((0,), (0,)), ((), ())),
                             preferred_element_type=jnp.float32)
        h2 = h2 + b2
        mean = jnp.mean(h2, axis=1, keepdims=True)
        ctr = h2 - mean
        var = jnp.mean(ctr * ctr, axis=1, keepdims=True)
        out_ref[pl.ds(s * V, V), :] = (
            ctr / jnp.sqrt(var + 1e-5) * gam + bet)


def _mlp(cv, W1, b1, W2, b2, ln_gamma, ln_beta):
    nt = B * V
    return pl.pallas_call(
        _mlp_body,
        grid=(B // RBLK,),
        in_specs=[
            pl.BlockSpec((RBLK, V), lambda i: (i, 0)),
            pl.BlockSpec((1, 128), lambda i: (0, 0)),
            pl.BlockSpec((128,), lambda i: (0,)),
            pl.BlockSpec((128, E), lambda i: (0, 0)),
            pl.BlockSpec((E,), lambda i: (0,)),
            pl.BlockSpec((E,), lambda i: (0,)),
            pl.BlockSpec((E,), lambda i: (0,)),
        ],
        out_specs=pl.BlockSpec((RBLK * V, E), lambda i: (i, 0)),
        out_shape=jax.ShapeDtypeStruct((nt, E), jnp.float32),
    )(cv, W1, b1, W2, b2, ln_gamma, ln_beta)


def kernel(feats_emb, center_coords, rand_scores, W1, b1, W2, b2,
           ln_gamma, ln_beta):
    posmap = _posmap(rand_scores)
    coords2d = center_coords.reshape(B, G)
    feats_flat = feats_emb.reshape(B * G, E)
    idx_all, cvis = _sc_compact(posmap, coords2d)
    fv = _sc_gather(idx_all, feats_flat)
    pos_emb = _mlp(cvis.reshape(B, V), W1, b1, W2, b2, ln_gamma, ln_beta)
    return fv, pos_emb.reshape(B, V, E)
